# Initial kernel scaffold; baseline (speedup 1.0000x reference)
import jax
import jax.numpy as jnp
from jax.experimental import pallas as pl


def kernel(x, p, edge_index, batch, W, b):
    raise NotImplementedError("write your pallas kernel here")


# speedup vs baseline: 23.6975x; 23.6975x over previous
import functools

import jax
import jax.numpy as jnp
from jax import lax
from jax.experimental import pallas as pl
from jax.experimental.pallas import tpu as pltpu
from jax.experimental.pallas import tpu_sc as plsc

N_NODES = 10000
N_EDGES = 320000
D_FEAT = 128
D_OUT = 32
N_GRAPHS = 64

NC = 2
NS = 16
LANES = 16

EPW = N_EDGES // (NC * NS)
CHUNK = 128
NCHUNK = (EPW + CHUNK - 1) // CHUNK
PADDED = NCHUNK * CHUNK
TBL = N_GRAPHS * N_NODES
TPW = TBL // NS
ZB = 10000


def _sc_body(src_hbm, dst_hbm, e_hbm, b_hbm, out_hbm,
             src_v, dst_v, e_v, bat_v, w_buf, f_buf, z_v, table):
    c = lax.axis_index("c")
    s = lax.axis_index("s")
    wid = c * NS + s
    base = wid * EPW

    pltpu.sync_copy(src_hbm.at[pl.ds(base, EPW)], src_v)
    pltpu.sync_copy(dst_hbm.at[pl.ds(base, EPW)], dst_v)
    pltpu.sync_copy(e_hbm, e_v)
    pltpu.sync_copy(b_hbm, bat_v)

    def zero_loop(i, _):
        z_v[pl.ds(pl.multiple_of(i * LANES, LANES), LANES)] = jnp.zeros(
            (LANES,), jnp.float32)
        return _
    lax.fori_loop(0, ZB // LANES, zero_loop, None)
    for k in range(TPW // ZB):
        pltpu.sync_copy(
            z_v, table.at[pl.ds(s * TPW + k * ZB, ZB)])

    for k in range((PADDED - EPW) // LANES):
        w_buf[NCHUNK - 1, pl.ds(EPW % CHUNK + k * LANES, LANES)] = jnp.zeros(
            (LANES,), jnp.float32)
        f_buf[NCHUNK - 1, pl.ds(EPW % CHUNK + k * LANES, LANES)] = jnp.zeros(
            (LANES,), jnp.int32)

    def edge_loop(i, _):
        off = pl.ds(pl.multiple_of(i * LANES, LANES), LANES)
        d = dst_v[off]
        sn = src_v[off]
        w = plsc.load_gather(e_v, [d])
        g = plsc.load_gather(bat_v, [d])
        f = g * N_NODES + sn
        row = i // (CHUNK // LANES)
        col = (i % (CHUNK // LANES)) * LANES
        w_buf[row, pl.ds(col, LANES)] = w
        f_buf[row, pl.ds(col, LANES)] = f
        return _
    lax.fori_loop(0, EPW // LANES, edge_loop, None)

    plsc.subcore_barrier()

    for j in range(NCHUNK):
        pltpu.sync_copy(w_buf.at[j], table.at[f_buf.at[j]], add=True)

    plsc.subcore_barrier()

    for k in range(TPW // ZB):
        pltpu.sync_copy(table.at[pl.ds(s * TPW + k * ZB, ZB)], z_v)
        pltpu.sync_copy(z_v, out_hbm.at[pl.ds(c * TBL + s * TPW + k * ZB, ZB)])


@functools.partial(jax.jit, static_argnums=())
def _sc_build_table(src, dst, e, bat):
    mesh = plsc.VectorSubcoreMesh(core_axis_name="c", subcore_axis_name="s")
    f = pl.kernel(
        _sc_body,
        out_type=jax.ShapeDtypeStruct((NC * TBL,), jnp.float32),
        mesh=mesh,
        compiler_params=pltpu.CompilerParams(needs_layout_passes=False),
        scratch_types=[
            pltpu.VMEM((EPW,), jnp.int32),
            pltpu.VMEM((EPW,), jnp.int32),
            pltpu.VMEM((N_NODES,), jnp.float32),
            pltpu.VMEM((N_NODES,), jnp.int32),
            pltpu.VMEM((NCHUNK, CHUNK), jnp.float32),
            pltpu.VMEM((NCHUNK, CHUNK), jnp.int32),
            pltpu.VMEM((ZB,), jnp.float32),
            pltpu.VMEM_SHARED((TBL,), jnp.float32),
        ],
    )
    return f(src, dst, e, bat)


def _tc_body(s_ref, x_ref, w_ref, b_ref, o_ref):
    st = s_ref[0] + s_ref[1]
    g = lax.dot(st, x_ref[...], preferred_element_type=jnp.float32)
    cnt = jnp.sum(st, axis=1, keepdims=True)
    o_ref[...] = (
        lax.dot(g, w_ref[...], preferred_element_type=jnp.float32)
        + cnt * b_ref[...])


def _tc_finish(s_tbl, x, w, b_row):
    return pl.pallas_call(
        _tc_body,
        out_shape=jax.ShapeDtypeStruct((N_GRAPHS, D_OUT), jnp.float32),
    )(s_tbl, x, w, b_row)


def kernel(x, p, edge_index, batch, W, b):
    src = edge_index[0].astype(jnp.int32)
    dst = edge_index[1].astype(jnp.int32)
    e = p[:, 0]
    bat = batch.astype(jnp.int32)
    s_tbl = _sc_build_table(src, dst, e, bat)
    s3 = s_tbl.reshape(NC, N_GRAPHS, N_NODES)
    return _tc_finish(s3, x, W, b.reshape(1, D_OUT))


# speedup vs baseline: 27.9829x; 1.1808x over previous
import functools

import jax
import jax.numpy as jnp
from jax import lax
from jax.experimental import pallas as pl
from jax.experimental.pallas import tpu as pltpu
from jax.experimental.pallas import tpu_sc as plsc

N_NODES = 10000
N_EDGES = 320000
D_FEAT = 128
D_OUT = 32
N_GRAPHS = 64

NC = 2
NS = 16
LANES = 16

EPW = N_EDGES // (NC * NS)
CHUNK = 128
NCHUNK = (EPW + CHUNK - 1) // CHUNK
PADDED = NCHUNK * CHUNK
TBL = N_GRAPHS * N_NODES
TPW = TBL // NS
ZB = 10000


def _sc_body(src_hbm, dst_hbm, e_hbm, b_hbm, out_hbm,
             src_v, dst_v, e_v, bat_v, w_buf, f_buf, z_v, y_v, table,
             in_sem, zt_sem, sc_sem, fl_sem):
    c = lax.axis_index("c")
    s = lax.axis_index("s")
    wid = c * NS + s
    base = wid * EPW

    in_cps = [
        pltpu.async_copy(src_hbm.at[pl.ds(base, EPW)], src_v, in_sem),
        pltpu.async_copy(dst_hbm.at[pl.ds(base, EPW)], dst_v, in_sem),
        pltpu.async_copy(e_hbm, e_v, in_sem),
        pltpu.async_copy(b_hbm, bat_v, in_sem),
    ]

    def zero_loop(i, _):
        z_v[pl.ds(pl.multiple_of(i * LANES, LANES), LANES)] = jnp.zeros(
            (LANES,), jnp.float32)
        return _
    lax.fori_loop(0, ZB // LANES, zero_loop, None)
    z_cps = [
        pltpu.async_copy(z_v, table.at[pl.ds(s * TPW + k * ZB, ZB)], zt_sem)
        for k in range(TPW // ZB)
    ]

    for k in range((PADDED - EPW) // LANES):
        w_buf[NCHUNK - 1, pl.ds(EPW % CHUNK + k * LANES, LANES)] = jnp.zeros(
            (LANES,), jnp.float32)
        f_buf[NCHUNK - 1, pl.ds(EPW % CHUNK + k * LANES, LANES)] = jnp.zeros(
            (LANES,), jnp.int32)

    for cp in in_cps:
        cp.wait()

    def edge_loop(i, _):
        off = pl.ds(pl.multiple_of(i * LANES, LANES), LANES)
        d = dst_v[off]
        sn = src_v[off]
        w = plsc.load_gather(e_v, [d])
        g = plsc.load_gather(bat_v, [d])
        f = g * N_NODES + sn
        row = i // (CHUNK // LANES)
        col = (i % (CHUNK // LANES)) * LANES
        w_buf[row, pl.ds(col, LANES)] = w
        f_buf[row, pl.ds(col, LANES)] = f
        return _
    lax.fori_loop(0, EPW // LANES, edge_loop, None)

    for cp in z_cps:
        cp.wait()
    plsc.subcore_barrier()

    sc_cps = [
        pltpu.async_copy(w_buf.at[j], table.at[f_buf.at[j]], sc_sem, add=True)
        for j in range(NCHUNK)
    ]
    for cp in sc_cps:
        cp.wait()

    plsc.subcore_barrier()

    bufs = [z_v, y_v]
    prev = None
    for k in range(TPW // ZB):
        buf = bufs[k % 2]
        pltpu.sync_copy(table.at[pl.ds(s * TPW + k * ZB, ZB)], buf)
        if prev is not None:
            prev.wait()
        prev = pltpu.async_copy(
            buf, out_hbm.at[pl.ds(c * TBL + s * TPW + k * ZB, ZB)], fl_sem)
    prev.wait()


@functools.partial(jax.jit, static_argnums=())
def _sc_build_table(src, dst, e, bat):
    mesh = plsc.VectorSubcoreMesh(core_axis_name="c", subcore_axis_name="s")
    f = pl.kernel(
        _sc_body,
        out_type=jax.ShapeDtypeStruct((NC * TBL,), jnp.float32),
        mesh=mesh,
        compiler_params=pltpu.CompilerParams(needs_layout_passes=False),
        scratch_types=[
            pltpu.VMEM((EPW,), jnp.int32),
            pltpu.VMEM((EPW,), jnp.int32),
            pltpu.VMEM((N_NODES,), jnp.float32),
            pltpu.VMEM((N_NODES,), jnp.int32),
            pltpu.VMEM((NCHUNK, CHUNK), jnp.float32),
            pltpu.VMEM((NCHUNK, CHUNK), jnp.int32),
            pltpu.VMEM((ZB,), jnp.float32),
            pltpu.VMEM((ZB,), jnp.float32),
            pltpu.VMEM_SHARED((TBL,), jnp.float32),
            pltpu.SemaphoreType.DMA,
            pltpu.SemaphoreType.DMA,
            pltpu.SemaphoreType.DMA,
            pltpu.SemaphoreType.DMA,
        ],
    )
    return f(src, dst, e, bat)


def _tc_body(s_ref, x_ref, w_ref, b_ref, o_ref):
    st = s_ref[0] + s_ref[1]
    g = lax.dot(st, x_ref[...], preferred_element_type=jnp.float32)
    cnt = jnp.sum(st, axis=1, keepdims=True)
    o_ref[...] = (
        lax.dot(g, w_ref[...], preferred_element_type=jnp.float32)
        + cnt * b_ref[...])


def _tc_finish(s_tbl, x, w, b_row):
    return pl.pallas_call(
        _tc_body,
        out_shape=jax.ShapeDtypeStruct((N_GRAPHS, D_OUT), jnp.float32),
    )(s_tbl, x, w, b_row)


def kernel(x, p, edge_index, batch, W, b):
    src = edge_index[0].astype(jnp.int32)
    dst = edge_index[1].astype(jnp.int32)
    e = p[:, 0]
    bat = batch.astype(jnp.int32)
    s_tbl = _sc_build_table(src, dst, e, bat)
    s3 = s_tbl.reshape(NC, N_GRAPHS, N_NODES)
    return _tc_finish(s3, x, W, b.reshape(1, D_OUT))


# speedup vs baseline: 40.0505x; 1.4312x over previous
import functools

import jax
import jax.numpy as jnp
from jax import lax
from jax.experimental import pallas as pl
from jax.experimental.pallas import tpu as pltpu
from jax.experimental.pallas import tpu_sc as plsc

N_NODES = 10000
N_EDGES = 320000
D_FEAT = 128
D_OUT = 32
N_GRAPHS = 64

NC = 2
NS = 16
LANES = 16

EPW = N_EDGES // (NC * NS)
CHUNK = 128
NCHUNK = (EPW + CHUNK - 1) // CHUNK
PADDED = NCHUNK * CHUNK
WLEN = PADDED
TBL = N_NODES * N_GRAPHS
TPW = TBL // NS
ZB = 10000


def _sc_body(edge_hbm, e_hbm, b_hbm, zero_hbm, out_hbm,
             ev, e_v, bat_v, w_buf, f_buf, z_v, y_v, table,
             in_sem, zt_sem, sc_sem, fl_sem):
    c = lax.axis_index("c")
    s = lax.axis_index("s")
    wid = c * NS + s
    base = wid * EPW

    base_al = jnp.minimum((base // 128) * 128, N_EDGES - WLEN)
    off0 = base - base_al
    z_cp = pltpu.async_copy(zero_hbm, z_v, zt_sem)
    in_cps = [
        pltpu.async_copy(
            edge_hbm.at[:, pl.ds(pl.multiple_of(base_al, 128), WLEN)],
            ev, in_sem),
        pltpu.async_copy(e_hbm, e_v, in_sem),
        pltpu.async_copy(b_hbm, bat_v, in_sem),
    ]

    z_cp.wait()
    z_cps = [
        pltpu.async_copy(z_v, table.at[pl.ds(s * TPW + k * ZB, ZB)], zt_sem)
        for k in range(TPW // ZB)
    ]

    for k in range((PADDED - EPW) // LANES):
        w_buf[NCHUNK - 1, pl.ds(EPW % CHUNK + k * LANES, LANES)] = jnp.zeros(
            (LANES,), jnp.float32)
        f_buf[NCHUNK - 1, pl.ds(EPW % CHUNK + k * LANES, LANES)] = jnp.zeros(
            (LANES,), jnp.int32)

    for cp in in_cps:
        cp.wait()

    @plsc.parallel_loop(0, EPW // LANES, step=1, unroll=8)
    def edge_loop(i):
        off = pl.ds(pl.multiple_of(off0 + i * LANES, LANES), LANES)
        sn = ev[0, off]
        d = ev[1, off]
        w = plsc.load_gather(e_v, [d])
        g = plsc.load_gather(bat_v, [d])
        f = sn * N_GRAPHS + g
        row = i // (CHUNK // LANES)
        col = (i % (CHUNK // LANES)) * LANES
        w_buf[row, pl.ds(col, LANES)] = w
        f_buf[row, pl.ds(col, LANES)] = f

    for cp in z_cps:
        cp.wait()
    plsc.subcore_barrier()

    sc_cps = [
        pltpu.async_copy(w_buf.at[j], table.at[f_buf.at[j]], sc_sem, add=True)
        for j in range(NCHUNK)
    ]
    for cp in sc_cps:
        cp.wait()

    plsc.subcore_barrier()

    bufs = [z_v, y_v]
    prev = None
    for k in range(TPW // ZB):
        buf = bufs[k % 2]
        pltpu.sync_copy(table.at[pl.ds(s * TPW + k * ZB, ZB)], buf)
        if prev is not None:
            prev.wait()
        prev = pltpu.async_copy(
            buf, out_hbm.at[pl.ds(c * TBL + s * TPW + k * ZB, ZB)], fl_sem)
    prev.wait()


@functools.partial(jax.jit, static_argnums=())
def _sc_build_table(edge_index, e, bat):
    mesh = plsc.VectorSubcoreMesh(core_axis_name="c", subcore_axis_name="s")
    f = pl.kernel(
        _sc_body,
        out_type=jax.ShapeDtypeStruct((NC * TBL,), jnp.float32),
        mesh=mesh,
        compiler_params=pltpu.CompilerParams(needs_layout_passes=False),
        scratch_types=[
            pltpu.VMEM((2, WLEN), jnp.int32),
            pltpu.VMEM((N_NODES,), jnp.float32),
            pltpu.VMEM((N_NODES,), jnp.int32),
            pltpu.VMEM((NCHUNK, CHUNK), jnp.float32),
            pltpu.VMEM((NCHUNK, CHUNK), jnp.int32),
            pltpu.VMEM((ZB,), jnp.float32),
            pltpu.VMEM((ZB,), jnp.float32),
            pltpu.VMEM_SHARED((TBL,), jnp.float32),
            pltpu.SemaphoreType.DMA,
            pltpu.SemaphoreType.DMA,
            pltpu.SemaphoreType.DMA,
            pltpu.SemaphoreType.DMA,
        ],
    )
    return f(edge_index, e, bat, jnp.zeros((ZB,), jnp.float32))


def _tc_body(s_ref, x_ref, w_ref, b_ref, o_ref):
    st = s_ref[0] + s_ref[1]
    m = lax.dot_general(st, x_ref[...], (((0,), (0,)), ((), ())),
                        preferred_element_type=jnp.float32)
    g = m[:N_GRAPHS, :D_FEAT] + m[N_GRAPHS:, D_FEAT:]
    cnt128 = jnp.sum(st, axis=0)
    cnt = cnt128[:N_GRAPHS] + cnt128[N_GRAPHS:]
    o_ref[...] = (
        lax.dot(g, w_ref[...], preferred_element_type=jnp.float32)
        + cnt[:, None] * b_ref[...])


def _tc_finish(s3, xr, w, b_row):
    return pl.pallas_call(
        _tc_body,
        out_shape=jax.ShapeDtypeStruct((N_GRAPHS, D_OUT), jnp.float32),
    )(s3, xr, w, b_row)


def kernel(x, p, edge_index, batch, W, b):
    e = p[:, 0]
    bat = batch.astype(jnp.int32)
    s_tbl = _sc_build_table(edge_index.astype(jnp.int32), e, bat)
    s3 = s_tbl.reshape(NC, N_NODES // 2, 2 * N_GRAPHS)
    xr = x.reshape(N_NODES // 2, 2 * D_FEAT)
    return _tc_finish(s3, xr, W, b.reshape(1, D_OUT))


# speedup vs baseline: 41.6677x; 1.0404x over previous
import functools

import jax
import jax.numpy as jnp
from jax import lax
from jax.experimental import pallas as pl
from jax.experimental.pallas import tpu as pltpu
from jax.experimental.pallas import tpu_sc as plsc

N_NODES = 10000
N_EDGES = 320000
D_FEAT = 128
D_OUT = 32
N_GRAPHS = 64

NC = 2
NS = 16
LANES = 16

EPW = N_EDGES // (NC * NS)
CHUNK = 128
NCHUNK = (EPW + CHUNK - 1) // CHUNK
PADDED = NCHUNK * CHUNK
WLEN = PADDED
TBL = N_NODES * N_GRAPHS
TPW = TBL // NS
ZB = 10000


def _sc_body(edge_hbm, pt_hbm, b_hbm, out_hbm,
             ev, e_v, bat_v, w_buf, f_buf, z_v, y_v, table,
             in_sem, zt_sem, sc_sem, fl_sem):
    c = lax.axis_index("c")
    s = lax.axis_index("s")
    wid = c * NS + s
    base = wid * EPW

    base_al = jnp.minimum((base // 128) * 128, N_EDGES - WLEN)
    off0 = base - base_al
    in_cps = [
        pltpu.async_copy(
            edge_hbm.at[:, pl.ds(pl.multiple_of(base_al, 128), WLEN)],
            ev, in_sem),
        pltpu.async_copy(pt_hbm.at[0], e_v, in_sem),
        pltpu.async_copy(b_hbm, bat_v, in_sem),
    ]

    with jax.named_scope("zero"):
        @plsc.parallel_loop(0, ZB // LANES, step=1, unroll=8)
        def zero_loop(i):
            z_v[pl.ds(pl.multiple_of(i * LANES, LANES), LANES)] = jnp.zeros(
                (LANES,), jnp.float32)
        z_cps = [
            pltpu.async_copy(z_v, table.at[pl.ds(s * TPW + k * ZB, ZB)],
                             zt_sem)
            for k in range(TPW // ZB)
        ]

        for k in range((PADDED - EPW) // LANES):
            w_buf[NCHUNK - 1, pl.ds(EPW % CHUNK + k * LANES, LANES)] = (
                jnp.zeros((LANES,), jnp.float32))
            f_buf[NCHUNK - 1, pl.ds(EPW % CHUNK + k * LANES, LANES)] = (
                jnp.zeros((LANES,), jnp.int32))

    with jax.named_scope("stage_wait"):
        for cp in in_cps:
            cp.wait()

    HALF = (EPW // LANES) // 2

    def edge_span(lo, hi):
        @plsc.parallel_loop(lo, hi, step=1, unroll=8)
        def edge_loop(i):
            off = pl.ds(pl.multiple_of(off0 + i * LANES, LANES), LANES)
            sn = ev[0, off]
            d = ev[1, off]
            w = plsc.load_gather(e_v, [d])
            g = plsc.load_gather(bat_v, [d])
            f = sn * N_GRAPHS + g
            row = i // (CHUNK // LANES)
            col = (i % (CHUNK // LANES)) * LANES
            w_buf[row, pl.ds(col, LANES)] = w
            f_buf[row, pl.ds(col, LANES)] = f

    ROWS1 = HALF // (CHUNK // LANES)
    with jax.named_scope("edges1"):
        edge_span(0, ROWS1 * (CHUNK // LANES))

    with jax.named_scope("zero_wait"):
        for cp in z_cps:
            cp.wait()
        plsc.subcore_barrier()

    with jax.named_scope("scatter1"):
        sc_cps = [
            pltpu.async_copy(w_buf.at[j], table.at[f_buf.at[j]], sc_sem,
                             add=True)
            for j in range(ROWS1)
        ]
    with jax.named_scope("edges2"):
        edge_span(ROWS1 * (CHUNK // LANES), EPW // LANES)
    with jax.named_scope("scatter2"):
        sc_cps += [
            pltpu.async_copy(w_buf.at[j], table.at[f_buf.at[j]], sc_sem,
                             add=True)
            for j in range(ROWS1, NCHUNK)
        ]
        for cp in sc_cps:
            cp.wait()

    plsc.subcore_barrier()

    bufs = [z_v, y_v]
    prev = None
    for k in range(TPW // ZB):
        buf = bufs[k % 2]
        pltpu.sync_copy(table.at[pl.ds(s * TPW + k * ZB, ZB)], buf)
        if prev is not None:
            prev.wait()
        prev = pltpu.async_copy(
            buf, out_hbm.at[pl.ds(c * TBL + s * TPW + k * ZB, ZB)], fl_sem)
    prev.wait()


@functools.partial(jax.jit, static_argnums=())
def _sc_build_table(edge_index, pt, bat):
    mesh = plsc.VectorSubcoreMesh(core_axis_name="c", subcore_axis_name="s")
    f = pl.kernel(
        _sc_body,
        out_type=jax.ShapeDtypeStruct((NC * TBL,), jnp.float32),
        mesh=mesh,
        compiler_params=pltpu.CompilerParams(needs_layout_passes=False),
        scratch_types=[
            pltpu.VMEM((2, WLEN), jnp.int32),
            pltpu.VMEM((N_NODES,), jnp.float32),
            pltpu.VMEM((N_NODES,), jnp.int32),
            pltpu.VMEM((NCHUNK, CHUNK), jnp.float32),
            pltpu.VMEM((NCHUNK, CHUNK), jnp.int32),
            pltpu.VMEM((ZB,), jnp.float32),
            pltpu.VMEM((ZB,), jnp.float32),
            pltpu.VMEM_SHARED((TBL,), jnp.float32),
            pltpu.SemaphoreType.DMA,
            pltpu.SemaphoreType.DMA,
            pltpu.SemaphoreType.DMA,
            pltpu.SemaphoreType.DMA,
        ],
    )
    return f(edge_index, pt, bat)


def _tc_body(s_ref, x_ref, w_ref, b_ref, o_ref):
    st = s_ref[0] + s_ref[1]
    m = lax.dot_general(st, x_ref[...], (((0,), (0,)), ((), ())),
                        preferred_element_type=jnp.float32)
    g = m[:N_GRAPHS, :D_FEAT] + m[N_GRAPHS:, D_FEAT:]
    cnt128 = jnp.sum(st, axis=0)
    cnt = cnt128[:N_GRAPHS] + cnt128[N_GRAPHS:]
    o_ref[...] = (
        lax.dot(g, w_ref[...], preferred_element_type=jnp.float32)
        + cnt[:, None] * b_ref[...])


def _tc_finish(s3, xr, w, b_row):
    return pl.pallas_call(
        _tc_body,
        out_shape=jax.ShapeDtypeStruct((N_GRAPHS, D_OUT), jnp.float32),
    )(s3, xr, w, b_row)


def kernel(x, p, edge_index, batch, W, b):
    pt = jnp.swapaxes(p, 0, 1)
    bat = batch.astype(jnp.int32)
    s_tbl = _sc_build_table(edge_index.astype(jnp.int32), pt, bat)
    s3 = s_tbl.reshape(NC, N_NODES // 2, 2 * N_GRAPHS)
    xr = x.reshape(N_NODES // 2, 2 * D_FEAT)
    return _tc_finish(s3, xr, W, b.reshape(1, D_OUT))
